# Initial kernel scaffold; baseline (speedup 1.0000x reference)
import jax
import jax.numpy as jnp
from jax.experimental import pallas as pl


def kernel(x, weight):
    raise NotImplementedError("write your pallas kernel here")


# speedup vs baseline: 2.9659x; 2.9659x over previous
import functools

import jax
import jax.numpy as jnp
from jax import lax
from jax.experimental import pallas as pl
from jax.experimental.pallas import tpu as pltpu
from jax.experimental.pallas import tpu_sc as plsc

EMBEDDING_DIM = 128
NUM_WORKERS = 32
CHUNK = 128


def _build_gather(batch):
    bpw = batch // NUM_WORKERS
    nch = bpw // CHUNK
    mesh = plsc.VectorSubcoreMesh(core_axis_name="c", subcore_axis_name="s")

    @functools.partial(
        pl.kernel,
        mesh=mesh,
        out_type=jax.ShapeDtypeStruct((batch, EMBEDDING_DIM), jnp.float32),
        scratch_types=[
            pltpu.VMEM((bpw,), jnp.int32),
            pltpu.VMEM((CHUNK, EMBEDDING_DIM), jnp.float32),
            pltpu.SemaphoreType.DMA,
        ],
    )
    def gather_kernel(table_hbm, idx_hbm, out_hbm, idx_v, rows_v, sem):
        wid = lax.axis_index("s") * 2 + lax.axis_index("c")
        base = wid * bpw
        pltpu.sync_copy(idx_hbm.at[pl.ds(base, bpw)], idx_v)

        def body(c, carry):
            off = c * CHUNK
            pltpu.async_copy(
                table_hbm.at[idx_v.at[pl.ds(off, CHUNK)]], rows_v, sem
            ).wait()
            pltpu.sync_copy(rows_v, out_hbm.at[pl.ds(base + off, CHUNK)])
            return carry

        lax.fori_loop(0, nch, body, 0)

    return gather_kernel


def kernel(x, weight):
    idx = x.reshape(-1).astype(jnp.int32)
    out = _build_gather(idx.shape[0])(weight, idx)
    return out.reshape(x.shape + (EMBEDDING_DIM,))


# speedup vs baseline: 3.3420x; 1.1268x over previous
import functools

import jax
import jax.numpy as jnp
from jax import lax
from jax.experimental import pallas as pl
from jax.experimental.pallas import tpu as pltpu
from jax.experimental.pallas import tpu_sc as plsc

EMBEDDING_DIM = 128
NUM_WORKERS = 32
CHUNK = 128
NBUF = 4


def _build_gather(batch):
    bpw = batch // NUM_WORKERS
    nch = bpw // CHUNK
    mesh = plsc.VectorSubcoreMesh(core_axis_name="c", subcore_axis_name="s")

    @functools.partial(
        pl.kernel,
        mesh=mesh,
        out_type=jax.ShapeDtypeStruct((batch, EMBEDDING_DIM), jnp.float32),
        scratch_types=[
            pltpu.VMEM((bpw,), jnp.int32),
            pltpu.VMEM((NBUF, CHUNK, EMBEDDING_DIM), jnp.float32),
            pltpu.SemaphoreType.DMA((NBUF,)),
            pltpu.SemaphoreType.DMA((NBUF,)),
        ],
    )
    def gather_kernel(table_hbm, idx_hbm, out_hbm, idx_v, rows_v, gsem, wsem):
        wid = lax.axis_index("s") * 2 + lax.axis_index("c")
        base = wid * bpw
        pltpu.sync_copy(idx_hbm.at[pl.ds(base, bpw)], idx_v)

        def start_gather(c, b):
            return pltpu.async_copy(
                table_hbm.at[idx_v.at[pl.ds(c * CHUNK, CHUNK)]],
                rows_v.at[b], gsem.at[b],
            )

        def wait_gather(c, b):
            pltpu.make_async_copy(
                table_hbm.at[idx_v.at[pl.ds(c * CHUNK, CHUNK)]],
                rows_v.at[b], gsem.at[b],
            ).wait()

        def start_write(c, b):
            return pltpu.async_copy(
                rows_v.at[b], out_hbm.at[pl.ds(base + c * CHUNK, CHUNK)],
                wsem.at[b],
            )

        def wait_write(c, b):
            pltpu.make_async_copy(
                rows_v.at[b], out_hbm.at[pl.ds(base + c * CHUNK, CHUNK)],
                wsem.at[b],
            ).wait()

        for c in range(NBUF - 1):
            start_gather(c, c % NBUF)
        start_gather(NBUF - 1, (NBUF - 1) % NBUF)
        wait_gather(0, 0)
        start_write(0, 0)

        n_steady = (nch - NBUF - 1) // NBUF

        def body(p, carry):
            for r in range(NBUF):
                c = 1 + p * NBUF + r
                b = (1 + r) % NBUF
                wait_write(c - 1, r % NBUF)
                start_gather(c + NBUF - 1, r % NBUF)
                wait_gather(c, b)
                start_write(c, b)
            return carry

        lax.fori_loop(0, n_steady, body, 0)

        first_tail = 1 + n_steady * NBUF
        for c in range(first_tail, nch):
            b = c % NBUF
            wait_write(c - 1, (c - 1) % NBUF)
            if c + NBUF - 1 < nch:
                start_gather(c + NBUF - 1, (c - 1) % NBUF)
            wait_gather(c, b)
            start_write(c, b)
        wait_write(nch - 1, (nch - 1) % NBUF)

    return gather_kernel


def kernel(x, weight):
    idx = x.reshape(-1).astype(jnp.int32)
    batch = idx.shape[0]
    out = _build_gather(batch)(weight, idx)
    return out.reshape(x.shape + (EMBEDDING_DIM,))


# speedup vs baseline: 5.9335x; 1.7754x over previous
import functools

import jax
import jax.numpy as jnp
from jax import lax
from jax.experimental import pallas as pl
from jax.experimental.pallas import tpu as pltpu
from jax.experimental.pallas import tpu_sc as plsc

EMBEDDING_DIM = 128
NUM_WORKERS = 32
ROWS_PER_CHUNK = 2
NBUF = 4


def _build_gather(nrows, nids):
    rpw = nrows // NUM_WORKERS
    nch = rpw // ROWS_PER_CHUNK
    mesh = plsc.VectorSubcoreMesh(core_axis_name="c", subcore_axis_name="s")

    @functools.partial(
        pl.kernel,
        mesh=mesh,
        out_type=jax.ShapeDtypeStruct((nrows, nids, EMBEDDING_DIM),
                                      jnp.float32),
        scratch_types=[
            pltpu.VMEM((rpw, nids), jnp.int32),
            pltpu.VMEM((NBUF, ROWS_PER_CHUNK, nids, EMBEDDING_DIM),
                       jnp.float32),
            pltpu.SemaphoreType.DMA((NBUF,)),
            pltpu.SemaphoreType.DMA((NBUF,)),
        ],
    )
    def gather_kernel(table_hbm, x_hbm, out_hbm, idx_v, rows_v, gsem, wsem):
        wid = lax.axis_index("s") * 2 + lax.axis_index("c")
        base = wid * rpw
        pltpu.sync_copy(x_hbm.at[pl.ds(base, rpw)], idx_v)

        def start_gather(c, b):
            for k in range(ROWS_PER_CHUNK):
                pltpu.async_copy(
                    table_hbm.at[idx_v.at[c * ROWS_PER_CHUNK + k]],
                    rows_v.at[b, k], gsem.at[b],
                )

        def wait_gather(c, b):
            for k in range(ROWS_PER_CHUNK):
                pltpu.make_async_copy(
                    table_hbm.at[idx_v.at[c * ROWS_PER_CHUNK + k]],
                    rows_v.at[b, k], gsem.at[b],
                ).wait()

        def start_write(c, b):
            pltpu.async_copy(
                rows_v.at[b],
                out_hbm.at[pl.ds(base + c * ROWS_PER_CHUNK, ROWS_PER_CHUNK)],
                wsem.at[b],
            )

        def wait_write(c, b):
            pltpu.make_async_copy(
                rows_v.at[b],
                out_hbm.at[pl.ds(base + c * ROWS_PER_CHUNK, ROWS_PER_CHUNK)],
                wsem.at[b],
            ).wait()

        for c in range(NBUF):
            start_gather(c, c % NBUF)
        wait_gather(0, 0)
        start_write(0, 0)

        n_steady = (nch - NBUF - 1) // NBUF

        def body(p, carry):
            for r in range(NBUF):
                c = 1 + p * NBUF + r
                b = (1 + r) % NBUF
                wait_write(c - 1, r % NBUF)
                start_gather(c + NBUF - 1, r % NBUF)
                wait_gather(c, b)
                start_write(c, b)
            return carry

        lax.fori_loop(0, n_steady, body, 0)

        first_tail = 1 + n_steady * NBUF
        for c in range(first_tail, nch):
            b = c % NBUF
            wait_write(c - 1, (c - 1) % NBUF)
            if c + NBUF - 1 < nch:
                start_gather(c + NBUF - 1, (c - 1) % NBUF)
            wait_gather(c, b)
            start_write(c, b)
        wait_write(nch - 1, (nch - 1) % NBUF)

    return gather_kernel


def kernel(x, weight):
    xi = x.astype(jnp.int32)
    return _build_gather(xi.shape[0], xi.shape[1])(weight, xi)
